# baseline (speedup 1.0000x reference)
import functools

import jax
import jax.numpy as jnp
from jax import lax
from jax.experimental import pallas as pl
from jax.experimental.pallas import tpu as pltpu
from jax.experimental.pallas import tpu_sc as plsc

N_ROWS = 320000
D_FEAT = 128
NUM_SEGMENTS = 10000
NV = D_FEAT // 16
NW = 32
SPT = 313
S_PAD = NW * SPT
CHUNK = 128
CD = CHUNK * D_FEAT
NBUF = 3
STAGE = SPT * D_FEAT


def _lane(vec, lane):
    idxv = (lax.iota(jnp.int32, 16) + lane) & 15
    return vec.at[idxv].get(mode="promise_in_bounds")[0]


def _search2(b_hbm, pa, pb, sema, semb, ta, tb):
    def probe_off(lo, hi):
        mid = jnp.maximum((lo + hi) // 2, 0)
        off = jnp.minimum((mid // 8) * 8, N_ROWS - 16)
        return mid, off

    def step(_, st):
        loa, hia, lob, hib = st
        mida, offa = probe_off(loa, hia)
        midb, offb = probe_off(lob, hib)
        cpa = pltpu.make_async_copy(b_hbm.at[pl.ds(offa, 16)], pa, sema)
        cpb = pltpu.make_async_copy(b_hbm.at[pl.ds(offb, 16)], pb, semb)
        cpa.start()
        cpb.start()
        cpa.wait()
        cpb.wait()
        gea = _lane(pa[...], mida - offa) >= ta
        geb = _lane(pb[...], midb - offb) >= tb
        return (jnp.where(gea, loa, mida), jnp.where(gea, mida, hia),
                jnp.where(geb, lob, midb), jnp.where(geb, midb, hib))

    init = (jnp.int32(-1), jnp.int32(N_ROWS), jnp.int32(-1), jnp.int32(N_ROWS))
    _, ra, _, rb = lax.fori_loop(0, 19, step, init)
    return ra, rb


def _body(xf_hbm, b_hbm, om_hbm, ox_hbm,
          xbuf, bbuf, pa, pb, mst, xst, xsem, bsem, sema, semb):
    wid = lax.axis_index("c") * 16 + lax.axis_index("s")
    seg_lo = wid * SPT

    r_lo, r_hi = _search2(b_hbm, pa, pb, sema, semb, seg_lo, seg_lo + SPT)

    off0 = jnp.minimum((r_lo // 8) * 8, N_ROWS - 16)
    cp0 = pltpu.make_async_copy(b_hbm.at[pl.ds(off0, 16)], pa, sema)
    cp0.start()
    cp0.wait()
    seg0 = _lane(pa[...], r_lo - off0)

    k_lo = r_lo // CHUNK
    k_hi = (r_hi + CHUNK - 1) // CHUNK
    full_start = (r_lo + CHUNK - 1) // CHUNK
    full_end = r_hi // CHUNK
    head_hi = jnp.minimum(full_start * CHUNK, r_hi)
    tail_lo = jnp.maximum(full_end * CHUNK, head_hi)

    def chunk_copies(k):
        p = k - (k // NBUF) * NBUF
        cpx = pltpu.make_async_copy(
            xf_hbm.at[pl.ds(k * CD, CD)],
            xbuf.at[pl.ds(p * CD, CD)],
            xsem.at[p])
        cpb = pltpu.make_async_copy(
            b_hbm.at[pl.ds(k * CHUNK, CHUNK)],
            bbuf.at[pl.ds(p * (CHUNK + 16), CHUNK)],
            bsem.at[p])
        return cpx, cpb

    def issue(k):
        cpx, cpb = chunk_copies(k)
        cpx.start()
        cpb.start()

    def drain(k):
        cpx, cpb = chunk_copies(k)
        cpx.wait()
        cpb.wait()

    for d in range(NBUF - 1):
        @pl.when(jnp.logical_and(r_lo < r_hi, k_lo + d < k_hi))
        def _(d=d):
            issue(k_lo + d)

    zero = jnp.zeros((16,), jnp.float32)

    def zstep(j, carry):
        mst[pl.ds(j * 16, 16)] = zero
        xst[pl.ds(j * 16, 16)] = zero
        return carry

    lax.fori_loop(0, SPT * NV, zstep, 0)

    def flush(seg_prev, cnt, sums, maxs):
        base = (seg_prev - seg_lo) * D_FEAT
        inv = 1.0 / jnp.full((16,), cnt, jnp.int32).astype(jnp.float32)
        for k in range(NV):
            mst[pl.ds(base + k * 16, 16)] = sums[k] * inv
            xst[pl.ds(base + k * 16, 16)] = maxs[k]

    def merge_row(seg, xk, carry):
        seg_prev, cnt = carry[0], carry[1]
        sums, maxs = carry[2:2 + NV], carry[2 + NV:]
        is_new = seg != seg_prev

        @pl.when(is_new)
        def _():
            flush(seg_prev, cnt, sums, maxs)

        keep = jnp.full((16,), jnp.where(is_new, 0.0, 1.0), jnp.float32)
        pen = keep * jnp.float32(3.4e38) - jnp.float32(3.4e38)
        nsums = [xk[k] + keep * sums[k] for k in range(NV)]
        nmaxs = [jnp.maximum(xk[k], maxs[k] * keep + pen) for k in range(NV)]
        ncnt = jnp.where(is_new, jnp.int32(1), cnt + 1)
        return (seg, ncnt) + tuple(nsums) + tuple(nmaxs)

    def row_dyn(k):
        p = k - (k // NBUF) * NBUF
        xoff = p * CD
        boff = p * (CHUNK + 16)

        def body(i, carry):
            seg = bbuf[pl.ds(boff + i, 16)][0]
            xk = [xbuf[pl.ds(xoff + i * D_FEAT + k2 * 16, 16)]
                  for k2 in range(NV)]
            return merge_row(seg, xk, carry)

        return body

    def chunk_body(k, carry):
        drain(k)

        @pl.when(k + NBUF - 1 < k_hi)
        def _():
            issue(k + NBUF - 1)

        p = k - (k // NBUF) * NBUF
        xoff = p * CD
        boff = p * (CHUNK + 16)

        def group_body(g, c):
            i0 = g * 16
            bv = bbuf[pl.ds(boff + i0, 16)]
            for j in range(16):
                xk = [xbuf[pl.ds(xoff + (i0 + j) * D_FEAT + k2 * 16, 16)]
                      for k2 in range(NV)]
                c = merge_row(bv[j], xk, c)
            return c

        return lax.fori_loop(0, CHUNK // 16, group_body, carry)

    neg = jnp.full((16,), -3.4e38, jnp.float32)
    carry = ((seg0, jnp.int32(0))
             + tuple(jnp.zeros((16,), jnp.float32) for _ in range(NV))
             + tuple(neg for _ in range(NV)))

    @pl.when(head_hi > r_lo)
    def _():
        drain(k_lo)

        @pl.when(k_lo + NBUF - 1 < k_hi)
        def _():
            issue(k_lo + NBUF - 1)

    carry = lax.fori_loop(r_lo - k_lo * CHUNK, head_hi - k_lo * CHUNK,
                          row_dyn(k_lo), carry)

    carry = lax.fori_loop(full_start, full_end, chunk_body, carry)

    @pl.when(r_hi > tail_lo)
    def _():
        drain(k_hi - 1)

    carry = lax.fori_loop(tail_lo - (k_hi - 1) * CHUNK,
                          r_hi - (k_hi - 1) * CHUNK,
                          row_dyn(k_hi - 1), carry)

    seg_prev, cnt = carry[0], carry[1]

    @pl.when(cnt > 0)
    def _():
        flush(seg_prev, cnt, carry[2:2 + NV], carry[2 + NV:])

    pltpu.sync_copy(mst, om_hbm.at[pl.ds(seg_lo * D_FEAT, STAGE)])
    pltpu.sync_copy(xst, ox_hbm.at[pl.ds(seg_lo * D_FEAT, STAGE)])


def _make_pool():
    return functools.partial(
        pl.kernel,
        out_type=[jax.ShapeDtypeStruct((S_PAD * D_FEAT,), jnp.float32),
                  jax.ShapeDtypeStruct((S_PAD * D_FEAT,), jnp.float32)],
        mesh=plsc.VectorSubcoreMesh(core_axis_name="c", subcore_axis_name="s"),
        scratch_types=[
            pltpu.VMEM((NBUF * CD,), jnp.float32),
            pltpu.VMEM((NBUF * (CHUNK + 16),), jnp.int32),
            pltpu.VMEM((16,), jnp.int32),
            pltpu.VMEM((16,), jnp.int32),
            pltpu.VMEM((STAGE,), jnp.float32),
            pltpu.VMEM((STAGE,), jnp.float32),
            pltpu.SemaphoreType.DMA((NBUF,)),
            pltpu.SemaphoreType.DMA((NBUF,)),
            pltpu.SemaphoreType.DMA,
            pltpu.SemaphoreType.DMA,
        ],
    )(_body)


def kernel(x, batch):
    om, ox = _make_pool()(x.reshape(-1), batch)
    mean = om.reshape(S_PAD, D_FEAT)[:NUM_SEGMENTS]
    mx = ox.reshape(S_PAD, D_FEAT)[:NUM_SEGMENTS]
    return jnp.concatenate([mean, mx], axis=-1)


# speedup vs baseline: 1.0386x; 1.0386x over previous
import functools

import jax
import jax.numpy as jnp
from jax import lax
from jax.experimental import pallas as pl
from jax.experimental.pallas import tpu as pltpu
from jax.experimental.pallas import tpu_sc as plsc

N_ROWS = 320000
D_FEAT = 128
NUM_SEGMENTS = 10000
NV = D_FEAT // 16
NW = 32
SPT = 313
S_PAD = NW * SPT
CHUNK = 128
CD = CHUNK * D_FEAT
NBUF = 3
STAGE = SPT * D_FEAT


def _lane(vec, lane):
    idxv = (lax.iota(jnp.int32, 16) + lane) & 15
    return vec.at[idxv].get(mode="promise_in_bounds")[0]


def _search2(b_hbm, pa, pb, sema, semb, ta, tb):
    def probe_off(lo, hi):
        mid = jnp.maximum((lo + hi) // 2, 0)
        off = jnp.minimum((mid // 8) * 8, N_ROWS - 16)
        return mid, off

    def step(_, st):
        loa, hia, lob, hib = st
        mida, offa = probe_off(loa, hia)
        midb, offb = probe_off(lob, hib)
        cpa = pltpu.make_async_copy(b_hbm.at[pl.ds(offa, 16)], pa, sema)
        cpb = pltpu.make_async_copy(b_hbm.at[pl.ds(offb, 16)], pb, semb)
        cpa.start()
        cpb.start()
        cpa.wait()
        cpb.wait()
        gea = _lane(pa[...], mida - offa) >= ta
        geb = _lane(pb[...], midb - offb) >= tb
        return (jnp.where(gea, loa, mida), jnp.where(gea, mida, hia),
                jnp.where(geb, lob, midb), jnp.where(geb, midb, hib))

    init = (jnp.int32(-1), jnp.int32(N_ROWS), jnp.int32(-1), jnp.int32(N_ROWS))
    _, ra, _, rb = lax.fori_loop(0, 19, step, init)
    return ra, rb


def _body(xf_hbm, b_hbm, om_hbm, ox_hbm,
          xbuf, bbuf, pa, pb, mst, xst, xsem, bsem, sema, semb):
    wid = lax.axis_index("c") * 16 + lax.axis_index("s")
    seg_lo = wid * SPT

    r_lo, r_hi = _search2(b_hbm, pa, pb, sema, semb, seg_lo, seg_lo + SPT)

    off0 = jnp.minimum((r_lo // 8) * 8, N_ROWS - 16)
    cp0 = pltpu.make_async_copy(b_hbm.at[pl.ds(off0, 16)], pa, sema)
    cp0.start()
    cp0.wait()
    seg0 = _lane(pa[...], r_lo - off0)

    k_lo = r_lo // CHUNK
    k_hi = (r_hi + CHUNK - 1) // CHUNK
    full_start = (r_lo + CHUNK - 1) // CHUNK
    full_end = r_hi // CHUNK
    head_hi = jnp.minimum(full_start * CHUNK, r_hi)
    tail_lo = jnp.maximum(full_end * CHUNK, head_hi)

    def chunk_copies(k):
        p = k - (k // NBUF) * NBUF
        cpx = pltpu.make_async_copy(
            xf_hbm.at[pl.ds(k * CD, CD)],
            xbuf.at[pl.ds(p * CD, CD)],
            xsem.at[p])
        cpb = pltpu.make_async_copy(
            b_hbm.at[pl.ds(k * CHUNK, CHUNK)],
            bbuf.at[pl.ds(p * (CHUNK + 16), CHUNK)],
            bsem.at[p])
        return cpx, cpb

    def issue(k):
        cpx, cpb = chunk_copies(k)
        cpx.start()
        cpb.start()

    def drain(k):
        cpx, cpb = chunk_copies(k)
        cpx.wait()
        cpb.wait()

    for d in range(NBUF - 1):
        @pl.when(jnp.logical_and(r_lo < r_hi, k_lo + d < k_hi))
        def _(d=d):
            issue(k_lo + d)

    zero = jnp.zeros((16,), jnp.float32)

    def zstep(j, carry):
        mst[pl.ds(j * 16, 16)] = zero
        xst[pl.ds(j * 16, 16)] = zero
        return carry

    lax.fori_loop(0, SPT * NV, zstep, 0)

    def flush_stores(base, cnt, sums, maxs):
        inv = 1.0 / jnp.full((16,), cnt, jnp.int32).astype(jnp.float32)
        for k in range(NV):
            mst[pl.ds(base + k * 16, 16)] = sums[k] * inv
            xst[pl.ds(base + k * 16, 16)] = maxs[k]

    def merge_row(seg, xk, carry):
        seg_prev, cnt = carry[0], carry[1]
        sums, maxs = carry[2:2 + NV], carry[2 + NV:]
        is_new = seg != seg_prev

        base = jnp.where(is_new, (seg_prev - seg_lo) * D_FEAT,
                         jnp.int32(SPT * D_FEAT))
        flush_stores(base, cnt, sums, maxs)

        keep = jnp.full((16,), jnp.where(is_new, 0.0, 1.0), jnp.float32)
        pen = keep * jnp.float32(3.4e38) - jnp.float32(3.4e38)
        nsums = [xk[k] + keep * sums[k] for k in range(NV)]
        nmaxs = [jnp.maximum(xk[k], maxs[k] * keep + pen) for k in range(NV)]
        ncnt = jnp.where(is_new, jnp.int32(1), cnt + 1)
        return (seg, ncnt) + tuple(nsums) + tuple(nmaxs)

    def row_dyn(k):
        p = k - (k // NBUF) * NBUF
        xoff = p * CD
        boff = p * (CHUNK + 16)

        def body(i, carry):
            seg = bbuf[pl.ds(boff + i, 16)][0]
            xk = [xbuf[pl.ds(xoff + i * D_FEAT + k2 * 16, 16)]
                  for k2 in range(NV)]
            return merge_row(seg, xk, carry)

        return body

    def chunk_body(k, carry):
        drain(k)

        @pl.when(k + NBUF - 1 < k_hi)
        def _():
            issue(k + NBUF - 1)

        p = k - (k // NBUF) * NBUF
        xoff = p * CD
        boff = p * (CHUNK + 16)

        def group_body(g, c):
            i0 = g * 16
            bv = bbuf[pl.ds(boff + i0, 16)]

            for j in range(16):
                xk = [xbuf[pl.ds(xoff + (i0 + j) * D_FEAT + k2 * 16, 16)]
                      for k2 in range(NV)]
                c = merge_row(bv[j], xk, c)
            return c

        return lax.fori_loop(0, CHUNK // 16, group_body, carry)

    neg = jnp.full((16,), -3.4e38, jnp.float32)
    carry = ((seg0, jnp.int32(0))
             + tuple(jnp.zeros((16,), jnp.float32) for _ in range(NV))
             + tuple(neg for _ in range(NV)))

    @pl.when(head_hi > r_lo)
    def _():
        drain(k_lo)

        @pl.when(k_lo + NBUF - 1 < k_hi)
        def _():
            issue(k_lo + NBUF - 1)

    carry = lax.fori_loop(r_lo - k_lo * CHUNK, head_hi - k_lo * CHUNK,
                          row_dyn(k_lo), carry)

    carry = lax.fori_loop(full_start, full_end, chunk_body, carry)

    @pl.when(r_hi > tail_lo)
    def _():
        drain(k_hi - 1)

    carry = lax.fori_loop(tail_lo - (k_hi - 1) * CHUNK,
                          r_hi - (k_hi - 1) * CHUNK,
                          row_dyn(k_hi - 1), carry)

    seg_prev, cnt = carry[0], carry[1]
    base_f = jnp.where(cnt > 0, (seg_prev - seg_lo) * D_FEAT,
                       jnp.int32(SPT * D_FEAT))
    flush_stores(base_f, cnt, carry[2:2 + NV], carry[2 + NV:])

    pltpu.sync_copy(mst.at[pl.ds(0, STAGE)],
                    om_hbm.at[pl.ds(seg_lo * D_FEAT, STAGE)])
    pltpu.sync_copy(xst.at[pl.ds(0, STAGE)],
                    ox_hbm.at[pl.ds(seg_lo * D_FEAT, STAGE)])


def _make_pool():
    return functools.partial(
        pl.kernel,
        out_type=[jax.ShapeDtypeStruct((S_PAD * D_FEAT,), jnp.float32),
                  jax.ShapeDtypeStruct((S_PAD * D_FEAT,), jnp.float32)],
        mesh=plsc.VectorSubcoreMesh(core_axis_name="c", subcore_axis_name="s"),
        scratch_types=[
            pltpu.VMEM((NBUF * CD,), jnp.float32),
            pltpu.VMEM((NBUF * (CHUNK + 16),), jnp.int32),
            pltpu.VMEM((16,), jnp.int32),
            pltpu.VMEM((16,), jnp.int32),
            pltpu.VMEM((STAGE + D_FEAT,), jnp.float32),
            pltpu.VMEM((STAGE + D_FEAT,), jnp.float32),
            pltpu.SemaphoreType.DMA((NBUF,)),
            pltpu.SemaphoreType.DMA((NBUF,)),
            pltpu.SemaphoreType.DMA,
            pltpu.SemaphoreType.DMA,
        ],
    )(_body)


def kernel(x, batch):
    om, ox = _make_pool()(x.reshape(-1), batch)
    mean = om.reshape(S_PAD, D_FEAT)[:NUM_SEGMENTS]
    mx = ox.reshape(S_PAD, D_FEAT)[:NUM_SEGMENTS]
    return jnp.concatenate([mean, mx], axis=-1)


# speedup vs baseline: 1.1156x; 1.0741x over previous
import functools

import jax
import jax.numpy as jnp
from jax import lax
from jax.experimental import pallas as pl
from jax.experimental.pallas import tpu as pltpu
from jax.experimental.pallas import tpu_sc as plsc

N_ROWS = 320000
D_FEAT = 128
NUM_SEGMENTS = 10000
NV = D_FEAT // 16
NW = 32
SPT = 313
S_PAD = NW * SPT
CHUNK = 128
CD = CHUNK * D_FEAT
NBUF = 3
STAGE = SPT * D_FEAT


def _lane(vec, lane):
    idxv = (lax.iota(jnp.int32, 16) + lane) & 15
    return vec.at[idxv].get(mode="promise_in_bounds")[0]


def _search2(b_hbm, pa, pb, sema, semb, ta, tb):
    def probe_off(lo, hi):
        mid = jnp.maximum((lo + hi) // 2, 0)
        off = jnp.minimum((mid // 8) * 8, N_ROWS - 16)
        return mid, off

    def step(_, st):
        loa, hia, lob, hib = st
        mida, offa = probe_off(loa, hia)
        midb, offb = probe_off(lob, hib)
        cpa = pltpu.make_async_copy(b_hbm.at[pl.ds(offa, 16)], pa, sema)
        cpb = pltpu.make_async_copy(b_hbm.at[pl.ds(offb, 16)], pb, semb)
        cpa.start()
        cpb.start()
        cpa.wait()
        cpb.wait()
        gea = _lane(pa[...], mida - offa) >= ta
        geb = _lane(pb[...], midb - offb) >= tb
        return (jnp.where(gea, loa, mida), jnp.where(gea, mida, hia),
                jnp.where(geb, lob, midb), jnp.where(geb, midb, hib))

    init = (jnp.int32(-1), jnp.int32(N_ROWS), jnp.int32(-1), jnp.int32(N_ROWS))
    _, ra, _, rb = lax.fori_loop(0, 19, step, init)
    return ra, rb


def _body(xf_hbm, b_hbm, om_hbm, ox_hbm,
          xbuf, bbuf, pa, pb, mst, xst, xsem, bsem, sema, semb):
    wid = lax.axis_index("c") * 16 + lax.axis_index("s")
    seg_lo = wid * SPT

    r_lo, r_hi = _search2(b_hbm, pa, pb, sema, semb, seg_lo, seg_lo + SPT)

    off0 = jnp.minimum((r_lo // 8) * 8, N_ROWS - 16)
    cp0 = pltpu.make_async_copy(b_hbm.at[pl.ds(off0, 16)], pa, sema)
    cp0.start()
    cp0.wait()
    seg0 = _lane(pa[...], r_lo - off0)

    k_lo = r_lo // CHUNK
    k_hi = (r_hi + CHUNK - 1) // CHUNK
    full_start = (r_lo + CHUNK - 1) // CHUNK
    full_end = r_hi // CHUNK
    head_hi = jnp.minimum(full_start * CHUNK, r_hi)
    tail_lo = jnp.maximum(full_end * CHUNK, head_hi)

    def chunk_copies(k):
        p = k - (k // NBUF) * NBUF
        cpx = pltpu.make_async_copy(
            xf_hbm.at[pl.ds(k * CD, CD)],
            xbuf.at[pl.ds(p * CD, CD)],
            xsem.at[p])
        cpb = pltpu.make_async_copy(
            b_hbm.at[pl.ds(k * CHUNK, CHUNK)],
            bbuf.at[pl.ds(p * (CHUNK + 16), CHUNK)],
            bsem.at[p])
        return cpx, cpb

    def issue(k):
        cpx, cpb = chunk_copies(k)
        cpx.start()
        cpb.start()

    def drain(k):
        cpx, cpb = chunk_copies(k)
        cpx.wait()
        cpb.wait()

    for d in range(NBUF - 1):
        @pl.when(jnp.logical_and(r_lo < r_hi, k_lo + d < k_hi))
        def _(d=d):
            issue(k_lo + d)

    zero = jnp.zeros((16,), jnp.float32)

    def zstep(j, carry):
        mst[pl.ds(j * 16, 16)] = zero
        xst[pl.ds(j * 16, 16)] = zero
        return carry

    lax.fori_loop(0, SPT * NV, zstep, 0)

    def flush_stores(base, cnt, sums, maxs):
        inv = 1.0 / jnp.full((16,), cnt, jnp.int32).astype(jnp.float32)
        for k in range(NV):
            mst[pl.ds(base + k * 16, 16)] = sums[k] * inv
            xst[pl.ds(base + k * 16, 16)] = maxs[k]

    def merge_nostore(seg, xk, carry):
        seg_prev, cnt = carry[0], carry[1]
        sums, maxs = carry[2:2 + NV], carry[2 + NV:]
        is_new = seg != seg_prev
        keep = jnp.full((16,), jnp.where(is_new, 0.0, 1.0), jnp.float32)
        pen = keep * jnp.float32(3.4e38) - jnp.float32(3.4e38)
        nsums = [xk[k] + keep * sums[k] for k in range(NV)]
        nmaxs = [jnp.maximum(xk[k], maxs[k] * keep + pen) for k in range(NV)]
        ncnt = jnp.where(is_new, jnp.int32(1), cnt + 1)
        return (seg, ncnt) + tuple(nsums) + tuple(nmaxs)

    def flush_addr(new_seg, carry):
        return jnp.where(new_seg != carry[0], (carry[0] - seg_lo) * D_FEAT,
                         jnp.int32(SPT * D_FEAT))

    def merge_row(seg, xk, carry):
        flush_stores(flush_addr(seg, carry), carry[1],
                     carry[2:2 + NV], carry[2 + NV:])
        return merge_nostore(seg, xk, carry)

    def row_dyn(k):
        p = k - (k // NBUF) * NBUF
        xoff = p * CD
        boff = p * (CHUNK + 16)

        def body(i, carry):
            seg = bbuf[pl.ds(boff + i, 16)][0]
            xk = [xbuf[pl.ds(xoff + i * D_FEAT + k2 * 16, 16)]
                  for k2 in range(NV)]
            return merge_row(seg, xk, carry)

        return body

    def chunk_body(k, carry):
        drain(k)

        @pl.when(k + NBUF - 1 < k_hi)
        def _():
            issue(k + NBUF - 1)

        p = k - (k // NBUF) * NBUF
        xoff = p * CD
        boff = p * (CHUNK + 16)

        def group_body(g, c):
            i0 = g * 16
            bv = bbuf[pl.ds(boff + i0, 16)]

            for j in range(0, 16, 2):
                xa = [xbuf[pl.ds(xoff + (i0 + j) * D_FEAT + k2 * 16, 16)]
                      for k2 in range(NV)]
                xb = [xbuf[pl.ds(xoff + (i0 + j + 1) * D_FEAT + k2 * 16, 16)]
                      for k2 in range(NV)]
                sa, sb = bv[j], bv[j + 1]
                c0 = c
                c1 = merge_nostore(sa, xa, c0)
                c = merge_nostore(sb, xb, c1)

                @pl.when(sb != c0[0])
                def _(sa=sa, sb=sb, c0=c0, c1=c1):
                    flush_stores(flush_addr(sa, c0), c0[1],
                                 c0[2:2 + NV], c0[2 + NV:])
                    flush_stores(flush_addr(sb, c1), c1[1],
                                 c1[2:2 + NV], c1[2 + NV:])
            return c

        return lax.fori_loop(0, CHUNK // 16, group_body, carry)

    neg = jnp.full((16,), -3.4e38, jnp.float32)
    carry = ((seg0, jnp.int32(0))
             + tuple(jnp.zeros((16,), jnp.float32) for _ in range(NV))
             + tuple(neg for _ in range(NV)))

    @pl.when(head_hi > r_lo)
    def _():
        drain(k_lo)

        @pl.when(k_lo + NBUF - 1 < k_hi)
        def _():
            issue(k_lo + NBUF - 1)

    carry = lax.fori_loop(r_lo - k_lo * CHUNK, head_hi - k_lo * CHUNK,
                          row_dyn(k_lo), carry)

    carry = lax.fori_loop(full_start, full_end, chunk_body, carry)

    @pl.when(r_hi > tail_lo)
    def _():
        drain(k_hi - 1)

    carry = lax.fori_loop(tail_lo - (k_hi - 1) * CHUNK,
                          r_hi - (k_hi - 1) * CHUNK,
                          row_dyn(k_hi - 1), carry)

    seg_prev, cnt = carry[0], carry[1]
    base_f = jnp.where(cnt > 0, (seg_prev - seg_lo) * D_FEAT,
                       jnp.int32(SPT * D_FEAT))
    flush_stores(base_f, cnt, carry[2:2 + NV], carry[2 + NV:])

    pltpu.sync_copy(mst.at[pl.ds(0, STAGE)],
                    om_hbm.at[pl.ds(seg_lo * D_FEAT, STAGE)])
    pltpu.sync_copy(xst.at[pl.ds(0, STAGE)],
                    ox_hbm.at[pl.ds(seg_lo * D_FEAT, STAGE)])


def _make_pool():
    return functools.partial(
        pl.kernel,
        out_type=[jax.ShapeDtypeStruct((S_PAD * D_FEAT,), jnp.float32),
                  jax.ShapeDtypeStruct((S_PAD * D_FEAT,), jnp.float32)],
        mesh=plsc.VectorSubcoreMesh(core_axis_name="c", subcore_axis_name="s"),
        scratch_types=[
            pltpu.VMEM((NBUF * CD,), jnp.float32),
            pltpu.VMEM((NBUF * (CHUNK + 16),), jnp.int32),
            pltpu.VMEM((16,), jnp.int32),
            pltpu.VMEM((16,), jnp.int32),
            pltpu.VMEM((STAGE + D_FEAT,), jnp.float32),
            pltpu.VMEM((STAGE + D_FEAT,), jnp.float32),
            pltpu.SemaphoreType.DMA((NBUF,)),
            pltpu.SemaphoreType.DMA((NBUF,)),
            pltpu.SemaphoreType.DMA,
            pltpu.SemaphoreType.DMA,
        ],
    )(_body)


def kernel(x, batch):
    om, ox = _make_pool()(x.reshape(-1), batch)
    mean = om.reshape(S_PAD, D_FEAT)[:NUM_SEGMENTS]
    mx = ox.reshape(S_PAD, D_FEAT)[:NUM_SEGMENTS]
    return jnp.concatenate([mean, mx], axis=-1)
